# baseline (speedup 1.0000x reference)
import jax
import jax.numpy as jnp
from jax.experimental import pallas as pl
from jax.experimental.pallas import tpu as pltpu

_INV_CHUNKS = 5
_EV_CHUNKS = 40


def _stream_actions(in_ref, out_ref, buf, isem, osem, chunks, rows):
    def mk_in(k):
        s = k % 2
        return pltpu.make_async_copy(in_ref.at[pl.ds(k * rows, rows)],
                                     buf.at[s], isem.at[s])

    def mk_out(k):
        s = k % 2
        return pltpu.make_async_copy(buf.at[s],
                                     out_ref.at[pl.ds(k * rows, rows)], osem.at[s])

    ins = [mk_in(k) for k in range(chunks)]
    outs = [mk_out(k) for k in range(chunks)]
    steps = []
    steps.append([ins[0].start])
    for k in range(chunks):
        acts = []
        if k + 1 < chunks:
            if k >= 1:
                acts.append(outs[k - 1].wait)
            acts.append(ins[k + 1].start)
        acts.append(ins[k].wait)
        acts.append(outs[k].start)
        steps.append(acts)
    tail = [outs[chunks - 1].wait]
    if chunks >= 2:
        tail.append(outs[chunks - 2].wait)
    steps.append(tail)
    return steps


def _copy_body(inv_in, ev_in, inv_out, ev_out, inv_buf, ev_buf,
               inv_isem, inv_osem, ev_isem, ev_osem):
    n, d = inv_in.shape
    ev_n = ev_in.shape[0]
    inv_steps = _stream_actions(inv_in, inv_out, inv_buf, inv_isem, inv_osem,
                                _INV_CHUNKS, n // _INV_CHUNKS)
    ev_steps = _stream_actions(ev_in, ev_out, ev_buf, ev_isem, ev_osem,
                               _EV_CHUNKS, ev_n // _EV_CHUNKS)
    ratio = max(1, len(ev_steps) // len(inv_steps))
    ii = 0
    for t, acts in enumerate(ev_steps):
        if t % ratio == 0 and ii < len(inv_steps):
            for a in inv_steps[ii]:
                a()
            ii += 1
        for a in acts:
            a()
    while ii < len(inv_steps):
        for a in inv_steps[ii]:
            a()
        ii += 1


def kernel(inv_features, ev_features, senders, receivers, sh_vectors, lengths,
           cutoffs, W1_inv, b1_inv, W2_inv, b2_inv, W1_ev, b1_ev, W2_ev, b2_ev):
    n, d_inv = inv_features.shape
    nn, s, d = ev_features.shape
    inv_out, ev_out = pl.pallas_call(
        _copy_body,
        in_specs=[
            pl.BlockSpec(memory_space=pl.ANY),
            pl.BlockSpec(memory_space=pl.ANY),
        ],
        out_specs=[
            pl.BlockSpec(memory_space=pl.ANY),
            pl.BlockSpec(memory_space=pl.ANY),
        ],
        out_shape=[
            jax.ShapeDtypeStruct(inv_features.shape, inv_features.dtype),
            jax.ShapeDtypeStruct(ev_features.shape, ev_features.dtype),
        ],
        scratch_shapes=[
            pltpu.VMEM((2, n // _INV_CHUNKS, d_inv), inv_features.dtype),
            pltpu.VMEM((2, nn // _EV_CHUNKS, s, d), ev_features.dtype),
            pltpu.SemaphoreType.DMA((2,)),
            pltpu.SemaphoreType.DMA((2,)),
            pltpu.SemaphoreType.DMA((2,)),
            pltpu.SemaphoreType.DMA((2,)),
        ],
    )(inv_features, ev_features)
    return (inv_out, ev_out)


# speedup vs baseline: 20.3932x; 20.3932x over previous
import jax
import jax.numpy as jnp
from jax.experimental import pallas as pl
from jax.experimental.pallas import tpu as pltpu

_STREAMS = 4
_CHUNKS_PER_STREAM = 2


def _copy_body(inv_in, inv_out, inv_buf, isems, osems):
    n, d = inv_in.shape
    total_chunks = _STREAMS * _CHUNKS_PER_STREAM
    rows = n // total_chunks

    def mk_in(st, k):
        s = k % 2
        chunk = k * _STREAMS + st
        return pltpu.make_async_copy(inv_in.at[pl.ds(chunk * rows, rows)],
                                     inv_buf.at[st, s], isems.at[st, s])

    def mk_out(st, k):
        s = k % 2
        chunk = k * _STREAMS + st
        return pltpu.make_async_copy(inv_buf.at[st, s],
                                     inv_out.at[pl.ds(chunk * rows, rows)],
                                     osems.at[st, s])

    ins = [[mk_in(st, k) for k in range(_CHUNKS_PER_STREAM)]
           for st in range(_STREAMS)]
    outs = [[mk_out(st, k) for k in range(_CHUNKS_PER_STREAM)]
            for st in range(_STREAMS)]
    for st in range(_STREAMS):
        ins[st][0].start()
    for k in range(_CHUNKS_PER_STREAM):
        for st in range(_STREAMS):
            if k + 1 < _CHUNKS_PER_STREAM:
                if k >= 1:
                    outs[st][k - 1].wait()
                ins[st][k + 1].start()
            ins[st][k].wait()
            outs[st][k].start()
    for st in range(_STREAMS):
        outs[st][_CHUNKS_PER_STREAM - 1].wait()
        if _CHUNKS_PER_STREAM >= 2:
            outs[st][_CHUNKS_PER_STREAM - 2].wait()


def kernel(inv_features, ev_features, senders, receivers, sh_vectors, lengths,
           cutoffs, W1_inv, b1_inv, W2_inv, b2_inv, W1_ev, b1_ev, W2_ev, b2_ev):
    n, d_inv = inv_features.shape
    rows = n // (_STREAMS * _CHUNKS_PER_STREAM)
    inv_out = pl.pallas_call(
        _copy_body,
        in_specs=[pl.BlockSpec(memory_space=pl.ANY)],
        out_specs=pl.BlockSpec(memory_space=pl.ANY),
        out_shape=jax.ShapeDtypeStruct(inv_features.shape, inv_features.dtype),
        scratch_shapes=[
            pltpu.VMEM((_STREAMS, 2, rows, d_inv), inv_features.dtype),
            pltpu.SemaphoreType.DMA((_STREAMS, 2)),
            pltpu.SemaphoreType.DMA((_STREAMS, 2)),
        ],
    )(inv_features)
    return (inv_out, ev_features)
